# baseline (speedup 1.0000x reference)
import functools

import jax
import jax.numpy as jnp
from jax import lax
from jax.experimental import pallas as pl
from jax.experimental.pallas import tpu as pltpu
from jax.experimental.pallas import tpu_sc as plsc

B = 32
C = 4
N = 24
L = 8
LAT = 48
LON = 96
NE = LAT * LON
HID = 128
OUT_LEN = 24
NC = 2
NS = 16
NW = NC * NS
KV = NE // 16


def _sc_argmin_body(clat_hbm, clon_hbm, csta_hbm,
                    j_out, dlat_out, dlon_out,
                    clat_v, clon_v, csta_v, jv, latv, lonv):
    wid = lax.axis_index("s") * NC + lax.axis_index("c")

    @pl.when(wid < N)
    def _():
        pltpu.sync_copy(clat_hbm, clat_v)
        pltpu.sync_copy(clon_hbm, clon_v)
        pltpu.sync_copy(csta_hbm, csta_v)
        slat = plsc.load_gather(
            csta_v, [jnp.full((16,), 2 * wid, jnp.int32)])
        slon = plsc.load_gather(
            csta_v, [jnp.full((16,), 2 * wid + 1, jnp.int32)])

        def body(k, carry):
            best, bidx = carry
            for u in range(4):
                off = k * 64 + u * 16
                cl = clat_v[pl.ds(off, 16)]
                cn = clon_v[pl.ds(off, 16)]
                dl = cl - slat
                dn = cn - slon
                d = dl * dl + dn * dn
                idx = lax.iota(jnp.int32, 16) + off
                upd = d < best
                best = jnp.where(upd, d, best)
                bidx = jnp.where(upd, idx, bidx)
            return best, bidx

        best, bidx = lax.fori_loop(
            0, KV // 4, body,
            (jnp.full((16,), 1e30, jnp.float32),
             jnp.zeros((16,), jnp.int32)))
        m = jnp.min(best)
        jm = jnp.min(jnp.where(best == m, bidx, jnp.int32(1 << 30)))
        jsplat = jnp.full((16,), jm, jnp.int32)
        jv[...] = jsplat
        latv[...] = plsc.load_gather(clat_v, [jsplat]) - slat
        lonv[...] = plsc.load_gather(clon_v, [jsplat]) - slon
        pltpu.sync_copy(jv, j_out.at[pl.ds(wid * 16, 16)])
        pltpu.sync_copy(latv, dlat_out.at[pl.ds(wid * 16, 16)])
        pltpu.sync_copy(lonv, dlon_out.at[pl.ds(wid * 16, 16)])


def _make_sc_argmin():
    return functools.partial(
        pl.kernel,
        out_type=(jax.ShapeDtypeStruct((NW * 16,), jnp.int32),
                  jax.ShapeDtypeStruct((NW * 16,), jnp.float32),
                  jax.ShapeDtypeStruct((NW * 16,), jnp.float32)),
        mesh=plsc.VectorSubcoreMesh(
            core_axis_name="c", subcore_axis_name="s",
            num_cores=NC, num_subcores=NS),
        compiler_params=pltpu.CompilerParams(needs_layout_passes=False),
        scratch_types=[
            pltpu.VMEM((NE,), jnp.float32),
            pltpu.VMEM((NE,), jnp.float32),
            pltpu.VMEM((2 * N,), jnp.float32),
            pltpu.VMEM((16,), jnp.int32),
            pltpu.VMEM((16,), jnp.float32),
            pltpu.VMEM((16,), jnp.float32),
        ])(_sc_argmin_body)


_OFF = {}
_off = 0
for _name, _rows, _pad in (
        ('eW1', 34, 40), ('eW2', 128, 128),
        ('m1W1', 194, 200), ('m1W2', 128, 128),
        ('u1W1', 256, 256), ('u1W2', 128, 128),
        ('m2W1', 194, 200), ('m2W2', 128, 128),
        ('u2W1', 256, 256), ('u2W2', 128, 128),
        ('oW1', 128, 128), ('oW2t', 24, 24), ('bias', 12, 16)):
    _OFF[_name] = _off
    _off += _pad
W_ROWS = _off
_BIASES = ('eb1', 'eb2', 'm1b1', 'm1b2', 'u1b1', 'u1b2',
           'm2b1', 'm2b2', 'u2b1', 'u2b2', 'ob1', 'ob2')


def _mlp_body(era_hbm, pan_hbm, j_smem, dlat_smem, dlon_smem, csta_smem,
              obs_r, w_r,
              out_ref, era_sc, pan_sc, sem):
    mm = lambda a, b: jnp.dot(a, b, preferred_element_type=jnp.float32)

    cps = []
    lons = []
    for n in range(N):
        j = j_smem[n * 16]
        lat = j // LON
        lons.append(j - lat * LON)
        cps.append(pltpu.async_copy(
            era_hbm.at[:, :, lat], era_sc.at[:, :, n], sem))
        cps.append(pltpu.async_copy(
            pan_hbm.at[:, :, lat], pan_sc.at[:, :, n], sem))

    def col24(vals, dtype):
        return jnp.concatenate(
            [jnp.full((1, 1), v, dtype) for v in vals], axis=0)

    def tile24(p):
        return jnp.broadcast_to(p[None], (B, N, HID)).reshape(B * N, HID)

    def row_of(W, r):
        oh = (lax.broadcasted_iota(jnp.int32, (1, W.shape[0]), 1)
              == r).astype(jnp.float32)
        return mm(oh, W)

    def psel(blk, cc):
        t = lax.broadcasted_iota(jnp.int32, (L, C * L), 0)
        r = lax.broadcasted_iota(jnp.int32, (L, C * L), 1)
        oh = (r == t * C + cc).astype(jnp.float32)
        return mm(oh, blk)

    clat = col24([csta_smem[n, 0] for n in range(N)], jnp.float32)
    clon = col24([csta_smem[n, 1] for n in range(N)], jnp.float32)
    dlat = col24([dlat_smem[n * 16] for n in range(N)], jnp.float32)
    dlon = col24([dlon_smem[n * 16] for n in range(N)], jnp.float32)

    obs4 = jnp.transpose(obs_r[...], (0, 1, 3, 2))
    obs = [obs4[:, cc].reshape(B * N, L) for cc in range(C)]

    W = w_r[...]
    blk = lambda name, k: W[_OFF[name]:_OFF[name] + k]
    bias = lambda name: W[_OFF['bias'] + _BIASES.index(name):
                          _OFF['bias'] + _BIASES.index(name) + 1]
    eW1 = blk('eW1', 34)
    eblk = eW1[0:C * L]
    acc = mm(obs[0], psel(eblk, 0))
    for cc in range(1, C):
        acc = acc + mm(obs[cc], psel(eblk, cc))
    p_emb = clon * row_of(eW1, C * L) + clat * row_of(eW1, C * L + 1)
    x = jnp.tanh(acc + tile24(p_emb) + bias('eb1'))
    x = jnp.tanh(mm(x, blk('eW2', HID)) + bias('eb2'))

    for cp in cps:
        cp.wait()
    lonrow = jnp.concatenate(
        [jnp.full((1, 1), lo, jnp.int32) for lo in lons], axis=1)
    I24r = jnp.broadcast_to(
        (lax.broadcasted_iota(jnp.int32, (N, N), 0)
         == lax.broadcasted_iota(jnp.int32, (N, N), 1)
         ).astype(jnp.float32)[None, :, None, :],
        (B, N, 1, N)).reshape(B * N, 1, N)

    def pick(sc_ref, width, cc):
        ohT = (lax.broadcasted_iota(jnp.int32, (width, N), 0)
               == lonrow).astype(jnp.float32)
        g = mm(sc_ref[:, cc].reshape(B * N * L, width), ohT)
        return jnp.sum(g.reshape(B * N, L, N) * I24r, axis=-1)

    era = [pick(era_sc, LON + 1, cc) for cc in range(C)]
    pan = [pick(pan_sc, LON, cc) for cc in range(C)]

    for ly in ('1', '2'):
        mW1 = blk('m%sW1' % ly, 194)
        eb = mW1[HID:HID + C * L]
        pb = mW1[HID + C * L:HID + 2 * C * L]
        acc = mm(x, mW1[0:HID])
        for cc in range(C):
            acc = acc + mm(era[cc], psel(eb, cc))
            acc = acc + mm(pan[cc], psel(pb, cc))
        p_pos = (dlon * row_of(mW1, HID + 2 * C * L)
                 + dlat * row_of(mW1, HID + 2 * C * L + 1))
        h = jnp.tanh(acc + tile24(p_pos) + bias('m%sb1' % ly))
        h = jnp.tanh(mm(h, blk('m%sW2' % ly, HID)) + bias('m%sb2' % ly))
        uW1 = blk('u%sW1' % ly, 2 * HID)
        o = jnp.tanh(mm(x, uW1[0:HID]) + mm(h, uW1[HID:2 * HID])
                     + bias('u%sb1' % ly))
        x = mm(o, blk('u%sW2' % ly, HID)) + bias('u%sb2' % ly)

    y = jnp.tanh(mm(x, blk('oW1', HID)) + bias('ob1'))
    out = lax.dot_general(
        y, blk('oW2t', OUT_LEN), (((1,), (1,)), ((), ())),
        preferred_element_type=jnp.float32) + bias('ob2')[:, 0:OUT_LEN]
    out_ref[...] = out.reshape(B, 1, N, OUT_LEN)


def kernel(obs_his, era_his, pan_fut, csta, cera, cpan,
           emb_W1, emb_b1, emb_W2, emb_b2,
           ex1_mW1, ex1_mb1, ex1_mW2, ex1_mb2,
           ex1_uW1, ex1_ub1, ex1_uW2, ex1_ub2,
           ex2_mW1, ex2_mb1, ex2_mW2, ex2_mb2,
           ex2_uW1, ex2_ub1, ex2_uW2, ex2_ub2,
           out_W1, out_b1, out_W2, out_b2):
    cand_lat = cera[:, :-1, 0].reshape(NE)
    cand_lon = cera[:, :-1, 1].reshape(NE)
    csta_flat = csta.reshape(2 * N)

    j_out, dlat_o, dlon_o = _make_sc_argmin()(cand_lat, cand_lon, csta_flat)

    era_t = jnp.swapaxes(era_his, 3, 4)
    pan_t = jnp.swapaxes(pan_fut, 3, 4)
    obs_t = jnp.swapaxes(obs_his, 2, 3)

    def pad_rows(a, r):
        return jnp.pad(a, ((0, r - a.shape[0]), (0, 0)))

    biases = jnp.stack(
        [emb_b1, emb_b2, ex1_mb1, ex1_mb2, ex1_ub1, ex1_ub2,
         ex2_mb1, ex2_mb2, ex2_ub1, ex2_ub2, out_b1,
         jnp.pad(out_b2, (0, HID - OUT_LEN))])
    wpack = jnp.concatenate([
        pad_rows(emb_W1, 40), emb_W2,
        pad_rows(ex1_mW1, 200), ex1_mW2, ex1_uW1, ex1_uW2,
        pad_rows(ex2_mW1, 200), ex2_mW2, ex2_uW1, ex2_uW2,
        out_W1, out_W2.T, pad_rows(biases, 16)], axis=0)

    hbm = pl.BlockSpec(memory_space=pltpu.HBM)
    smem = pl.BlockSpec(memory_space=pltpu.SMEM)
    vmem = pl.BlockSpec(memory_space=pltpu.VMEM)
    args = [era_t, pan_t, j_out, dlat_o, dlon_o, csta,
            obs_t, wpack]
    in_specs = [hbm, hbm, smem, smem, smem, smem, vmem, vmem]

    return pl.pallas_call(
        _mlp_body,
        out_shape=jax.ShapeDtypeStruct((B, 1, N, OUT_LEN), jnp.float32),
        in_specs=in_specs,
        out_specs=vmem,
        compiler_params=pltpu.CompilerParams(
            vmem_limit_bytes=56 * 1024 * 1024),
        scratch_shapes=[
            pltpu.VMEM((B, C, N, L, LON + 1), jnp.float32),
            pltpu.VMEM((B, C, N, L, LON), jnp.float32),
            pltpu.SemaphoreType.DMA,
        ],
    )(*args)


# speedup vs baseline: 1.2045x; 1.2045x over previous
import functools

import jax
import jax.numpy as jnp
from jax import lax
from jax.experimental import pallas as pl
from jax.experimental.pallas import tpu as pltpu
from jax.experimental.pallas import tpu_sc as plsc

B = 32
C = 4
N = 24
L = 8
LAT = 48
LON = 96
NE = LAT * LON
HID = 128
OUT_LEN = 24
NC = 2
NS = 16
NW = NC * NS
KV = NE // 16


def _sc_argmin_body(clat_hbm, clon_hbm, csta_hbm,
                    j_out, dlat_out, dlon_out,
                    clat_v, clon_v, csta_v, jv, latv, lonv):
    wid = lax.axis_index("s") * NC + lax.axis_index("c")

    @pl.when(wid < N)
    def _():
        pltpu.sync_copy(clat_hbm, clat_v)
        pltpu.sync_copy(clon_hbm, clon_v)
        pltpu.sync_copy(csta_hbm, csta_v)
        slat = plsc.load_gather(
            csta_v, [jnp.full((16,), 2 * wid, jnp.int32)])
        slon = plsc.load_gather(
            csta_v, [jnp.full((16,), 2 * wid + 1, jnp.int32)])

        def body(k, carry):
            best, bidx = carry
            for u in range(4):
                off = k * 64 + u * 16
                cl = clat_v[pl.ds(off, 16)]
                cn = clon_v[pl.ds(off, 16)]
                dl = cl - slat
                dn = cn - slon
                d = dl * dl + dn * dn
                idx = lax.iota(jnp.int32, 16) + off
                upd = d < best
                best = jnp.where(upd, d, best)
                bidx = jnp.where(upd, idx, bidx)
            return best, bidx

        best, bidx = lax.fori_loop(
            0, KV // 4, body,
            (jnp.full((16,), 1e30, jnp.float32),
             jnp.zeros((16,), jnp.int32)))
        m = jnp.min(best)
        jm = jnp.min(jnp.where(best == m, bidx, jnp.int32(1 << 30)))
        jsplat = jnp.full((16,), jm, jnp.int32)
        jv[...] = jsplat
        latv[...] = plsc.load_gather(clat_v, [jsplat]) - slat
        lonv[...] = plsc.load_gather(clon_v, [jsplat]) - slon
        pltpu.sync_copy(jv, j_out.at[pl.ds(wid * 16, 16)])
        pltpu.sync_copy(latv, dlat_out.at[pl.ds(wid * 16, 16)])
        pltpu.sync_copy(lonv, dlon_out.at[pl.ds(wid * 16, 16)])


def _make_sc_argmin():
    return functools.partial(
        pl.kernel,
        out_type=(jax.ShapeDtypeStruct((NW * 16,), jnp.int32),
                  jax.ShapeDtypeStruct((NW * 16,), jnp.float32),
                  jax.ShapeDtypeStruct((NW * 16,), jnp.float32)),
        mesh=plsc.VectorSubcoreMesh(
            core_axis_name="c", subcore_axis_name="s",
            num_cores=NC, num_subcores=NS),
        compiler_params=pltpu.CompilerParams(needs_layout_passes=False),
        scratch_types=[
            pltpu.VMEM((NE,), jnp.float32),
            pltpu.VMEM((NE,), jnp.float32),
            pltpu.VMEM((2 * N,), jnp.float32),
            pltpu.VMEM((16,), jnp.int32),
            pltpu.VMEM((16,), jnp.float32),
            pltpu.VMEM((16,), jnp.float32),
        ])(_sc_argmin_body)


def _mlp_body(era_hbm, pan_hbm, j_smem, dlat_smem, dlon_smem, csta_smem,
              obs_r,
              eW1_r, eb1_r, eW2_r, eb2_r,
              m1W1_r, m1b1_r, m1W2_r, m1b2_r, u1W1_r, u1b1_r, u1W2_r, u1b2_r,
              m2W1_r, m2b1_r, m2W2_r, m2b2_r, u2W1_r, u2b1_r, u2W2_r, u2b2_r,
              oW1_r, ob1_r, oW2t_r, ob2_r,
              out_ref, era_sc, pan_sc, sem):
    mm = lambda a, b: jnp.dot(a, b, preferred_element_type=jnp.float32)

    cps = []
    lons = []
    for n in range(N):
        j = j_smem[n * 16]
        lat = j // LON
        lons.append(j - lat * LON)
        cps.append(pltpu.async_copy(
            era_hbm.at[:, :, lat], era_sc.at[:, :, n], sem))
        cps.append(pltpu.async_copy(
            pan_hbm.at[:, :, lat], pan_sc.at[:, :, n], sem))

    def col24(vals, dtype):
        return jnp.concatenate(
            [jnp.full((1, 1), v, dtype) for v in vals], axis=0)

    def tile24(p):
        return jnp.broadcast_to(p[None], (B, N, HID)).reshape(B * N, HID)

    def row_of(W, r):
        oh = (lax.broadcasted_iota(jnp.int32, (1, W.shape[0]), 1)
              == r).astype(jnp.float32)
        return mm(oh, W)

    def psel(blk, cc):
        t = lax.broadcasted_iota(jnp.int32, (L, C * L), 0)
        r = lax.broadcasted_iota(jnp.int32, (L, C * L), 1)
        oh = (r == t * C + cc).astype(jnp.float32)
        return mm(oh, blk)

    clat = col24([csta_smem[n, 0] for n in range(N)], jnp.float32)
    clon = col24([csta_smem[n, 1] for n in range(N)], jnp.float32)
    dlat = col24([dlat_smem[n * 16] for n in range(N)], jnp.float32)
    dlon = col24([dlon_smem[n * 16] for n in range(N)], jnp.float32)

    obs4 = jnp.transpose(obs_r[...], (0, 1, 3, 2))
    obs = [obs4[:, cc].reshape(B * N, L) for cc in range(C)]

    eW1 = eW1_r[...]
    eblk = eW1[0:C * L]
    acc = mm(obs[0], psel(eblk, 0))
    for cc in range(1, C):
        acc = acc + mm(obs[cc], psel(eblk, cc))
    p_emb = clon * row_of(eW1, C * L) + clat * row_of(eW1, C * L + 1)
    x = jnp.tanh(acc + tile24(p_emb) + eb1_r[...].reshape(1, -1))
    x = jnp.tanh(mm(x, eW2_r[...]) + eb2_r[...].reshape(1, -1))

    for cp in cps:
        cp.wait()
    lonrow = jnp.concatenate(
        [jnp.full((1, 1), lo, jnp.int32) for lo in lons], axis=1)
    I24r = jnp.broadcast_to(
        (lax.broadcasted_iota(jnp.int32, (N, N), 0)
         == lax.broadcasted_iota(jnp.int32, (N, N), 1)
         ).astype(jnp.float32)[None, :, None, :],
        (B, N, 1, N)).reshape(B * N, 1, N)

    def pick(sc_ref, width, cc):
        ohT = (lax.broadcasted_iota(jnp.int32, (width, N), 0)
               == lonrow).astype(jnp.float32)
        g = mm(sc_ref[:, cc].reshape(B * N * L, width), ohT)
        return jnp.sum(g.reshape(B * N, L, N) * I24r, axis=-1)

    era = [pick(era_sc, LON + 1, cc) for cc in range(C)]
    pan = [pick(pan_sc, LON, cc) for cc in range(C)]

    for (mW1_r, mb1_r, mW2_r, mb2_r, uW1_r, ub1_r, uW2_r, ub2_r) in (
            (m1W1_r, m1b1_r, m1W2_r, m1b2_r, u1W1_r, u1b1_r, u1W2_r, u1b2_r),
            (m2W1_r, m2b1_r, m2W2_r, m2b2_r, u2W1_r, u2b1_r, u2W2_r, u2b2_r)):
        mW1 = mW1_r[...]
        eb = mW1[HID:HID + C * L]
        pb = mW1[HID + C * L:HID + 2 * C * L]
        acc = mm(x, mW1[0:HID])
        for cc in range(C):
            acc = acc + mm(era[cc], psel(eb, cc))
            acc = acc + mm(pan[cc], psel(pb, cc))
        p_pos = (dlon * row_of(mW1, HID + 2 * C * L)
                 + dlat * row_of(mW1, HID + 2 * C * L + 1))
        h = jnp.tanh(acc + tile24(p_pos) + mb1_r[...].reshape(1, -1))
        h = jnp.tanh(mm(h, mW2_r[...]) + mb2_r[...].reshape(1, -1))
        uW1 = uW1_r[...]
        o = jnp.tanh(mm(x, uW1[0:HID]) + mm(h, uW1[HID:2 * HID])
                     + ub1_r[...].reshape(1, -1))
        x = mm(o, uW2_r[...]) + ub2_r[...].reshape(1, -1)

    y = jnp.tanh(mm(x, oW1_r[...]) + ob1_r[...].reshape(1, -1))
    out = lax.dot_general(
        y, oW2t_r[...], (((1,), (1,)), ((), ())),
        preferred_element_type=jnp.float32) + ob2_r[...].reshape(1, -1)
    out_ref[...] = out.reshape(B, 1, N, OUT_LEN)


def kernel(obs_his, era_his, pan_fut, csta, cera, cpan,
           emb_W1, emb_b1, emb_W2, emb_b2,
           ex1_mW1, ex1_mb1, ex1_mW2, ex1_mb2,
           ex1_uW1, ex1_ub1, ex1_uW2, ex1_ub2,
           ex2_mW1, ex2_mb1, ex2_mW2, ex2_mb2,
           ex2_uW1, ex2_ub1, ex2_uW2, ex2_ub2,
           out_W1, out_b1, out_W2, out_b2):
    cand_lat = cera[:, :-1, 0].reshape(NE)
    cand_lon = cera[:, :-1, 1].reshape(NE)
    csta_flat = csta.reshape(2 * N)

    j_out, dlat_o, dlon_o = _make_sc_argmin()(cand_lat, cand_lon, csta_flat)

    era_t = jnp.swapaxes(era_his, 3, 4)
    pan_t = jnp.swapaxes(pan_fut, 3, 4)
    obs_t = jnp.swapaxes(obs_his, 2, 3)
    oW2t = out_W2.T

    hbm = pl.BlockSpec(memory_space=pltpu.HBM)
    smem = pl.BlockSpec(memory_space=pltpu.SMEM)
    vmem = pl.BlockSpec(memory_space=pltpu.VMEM)
    args = [era_t, pan_t, j_out, dlat_o, dlon_o, csta,
            obs_t,
            emb_W1, emb_b1, emb_W2, emb_b2,
            ex1_mW1, ex1_mb1, ex1_mW2, ex1_mb2,
            ex1_uW1, ex1_ub1, ex1_uW2, ex1_ub2,
            ex2_mW1, ex2_mb1, ex2_mW2, ex2_mb2,
            ex2_uW1, ex2_ub1, ex2_uW2, ex2_ub2,
            out_W1, out_b1, oW2t, out_b2]
    in_specs = [hbm, hbm, smem, smem, smem, smem] + [vmem] * (len(args) - 6)

    return pl.pallas_call(
        _mlp_body,
        out_shape=jax.ShapeDtypeStruct((B, 1, N, OUT_LEN), jnp.float32),
        in_specs=in_specs,
        out_specs=vmem,
        compiler_params=pltpu.CompilerParams(
            vmem_limit_bytes=56 * 1024 * 1024),
        scratch_shapes=[
            pltpu.VMEM((B, C, N, L, LON + 1), jnp.float32),
            pltpu.VMEM((B, C, N, L, LON), jnp.float32),
            pltpu.SemaphoreType.DMA,
        ],
    )(*args)
